# baseline (speedup 1.0000x reference)
from math import sqrt

import jax
import jax.numpy as jnp
from jax import lax
from jax.experimental import pallas as pl
from jax.experimental.pallas import tpu as pltpu
from jax.experimental.pallas import tpu_sc as plsc

D_MODEL = 1024
SCALE = sqrt(D_MODEL)
NW = 32
LANES = 16
PCHUNK = 8
S_SC = 1024
K_TC = 128


def _make_sc_kernel(batch, seq):
    pos_per_w = S_SC // NW
    n_chunks = pos_per_w // PCHUNK
    mesh = plsc.VectorSubcoreMesh(core_axis_name="c", subcore_axis_name="s")

    @pl.kernel(
        out_type=jax.ShapeDtypeStruct((batch * seq, D_MODEL), jnp.float32),
        mesh=mesh,
        scratch_types=[
            pltpu.VMEM((batch * pos_per_w,), jnp.int32),
            pltpu.VMEM((batch, PCHUNK, D_MODEL), jnp.float32),
            pltpu.VMEM((batch, PCHUNK, D_MODEL), jnp.float32),
            pltpu.VMEM((batch, PCHUNK, D_MODEL), jnp.float32),
            pltpu.VMEM((PCHUNK, D_MODEL), jnp.float32),
            pltpu.VMEM((PCHUNK, D_MODEL), jnp.float32),
            pltpu.VMEM((PCHUNK, D_MODEL), jnp.float32),
            pltpu.SemaphoreType.DMA,
            pltpu.SemaphoreType.DMA,
            pltpu.SemaphoreType.DMA,
            pltpu.SemaphoreType.DMA,
        ],
    )
    def emb_kernel(
        x_hbm, table_hbm, pe_hbm, out_hbm,
        idx_v, rows0, rows1, rows2, pe0, pe1, pe2,
        sem0, sem1, sem2, sem_out,
    ):
        rows = (rows0, rows1, rows2)
        pes = (pe0, pe1, pe2)
        sems = (sem0, sem1, sem2)

        wid = lax.axis_index("s") * 2 + lax.axis_index("c")
        p0 = wid * pos_per_w

        idx_copies = [
            pltpu.async_copy(
                x_hbm.at[pl.ds(b * seq + p0, pos_per_w)],
                idx_v.at[pl.ds(b * pos_per_w, pos_per_w)],
                sem0,
            )
            for b in range(batch)
        ]
        for cp in idx_copies:
            cp.wait()

        def fire_in(c, k):
            off = c * PCHUNK
            cps = [
                pltpu.async_copy(
                    table_hbm.at[idx_v.at[pl.ds(b * pos_per_w + off, PCHUNK)]],
                    rows[k].at[b],
                    sems[k],
                )
                for b in range(batch)
            ]
            cps.append(
                pltpu.async_copy(pe_hbm.at[pl.ds(p0 + off, PCHUNK)], pes[k], sems[k])
            )
            return cps

        def fire_stores(c, k):
            off = c * PCHUNK
            return [
                pltpu.async_copy(
                    rows[k].at[b],
                    out_hbm.at[pl.ds(b * seq + p0 + off, PCHUNK)],
                    sem_out,
                )
                for b in range(batch)
            ]

        def compute(k):
            def row_body(r, carry):
                def col_body(j, carry2):
                    sl = pl.ds(j * LANES, LANES)
                    pe_reg = pes[k][r, sl]
                    for b in range(batch):
                        rows[k][b, r, sl] = rows[k][b, r, sl] * SCALE + pe_reg
                    return carry2

                return lax.fori_loop(
                    0, D_MODEL // LANES, col_body, carry, unroll=8
                )

            lax.fori_loop(0, PCHUNK, row_body, 0)

        nbuf = 3
        in_flight = {0: fire_in(0, 0), 1: fire_in(1, 1)}
        store_flight = {}
        for c in range(n_chunks):
            k = c % nbuf
            if c + 2 < n_chunks:
                for cp in store_flight.pop(c - 1, ()):
                    cp.wait()
                in_flight[c + 2] = fire_in(c + 2, (c + 2) % nbuf)
            for cp in in_flight.pop(c):
                cp.wait()
            compute(k)
            store_flight[c] = fire_stores(c, k)
        for cps in store_flight.values():
            for cp in cps:
                cp.wait()

    return emb_kernel


def _make_tc_kernel(batch, seq):
    n_pos = seq - S_SC
    steps = n_pos // K_TC
    n_steps = batch * steps

    def body(idx_sm, table_hbm, pe_vm, out_vm, buf, sem0, sem1):
        b = pl.program_id(0)
        i = pl.program_id(1)
        t = b * steps + i
        sems = (sem0, sem1)

        def fire(t2, slot):
            b2 = t2 // steps
            i2 = t2 % steps
            base = b2 * seq + S_SC + i2 * K_TC
            for j in range(K_TC):
                pltpu.make_async_copy(
                    table_hbm.at[pl.ds(idx_sm[base + j], 1)],
                    buf.at[pl.ds(slot * K_TC + j, 1)],
                    sems[slot],
                ).start()

        def drain(slot):
            for j in range(K_TC):
                pltpu.make_async_copy(
                    table_hbm.at[pl.ds(0, 1)],
                    buf.at[pl.ds(slot * K_TC + j, 1)],
                    sems[slot],
                ).wait()

        parity = t % 2

        @pl.when(t == 0)
        def _prime():
            fire(t, 0)

        @pl.when((t + 1 < n_steps) & (parity == 0))
        def _prefetch_odd():
            fire(t + 1, 1)

        @pl.when((t + 1 < n_steps) & (parity == 1))
        def _prefetch_even():
            fire(t + 1, 0)

        @pl.when(parity == 0)
        def _drain0():
            drain(0)

        @pl.when(parity == 1)
        def _drain1():
            drain(1)

        out_vm[0] = buf[pl.ds(parity * K_TC, K_TC)] * SCALE + pe_vm[...]

    grid_spec = pltpu.PrefetchScalarGridSpec(
        num_scalar_prefetch=1,
        grid=(batch, steps),
        in_specs=[
            pl.BlockSpec(memory_space=pltpu.MemorySpace.HBM),
            pl.BlockSpec((K_TC, D_MODEL), lambda b, i, idx: (S_SC // K_TC + i, 0)),
        ],
        out_specs=pl.BlockSpec((1, K_TC, D_MODEL), lambda b, i, idx: (b, i, 0)),
        scratch_shapes=[
            pltpu.VMEM((2 * K_TC, D_MODEL), jnp.float32),
            pltpu.SemaphoreType.DMA,
            pltpu.SemaphoreType.DMA,
        ],
    )
    return pl.pallas_call(
        body,
        grid_spec=grid_spec,
        out_shape=jax.ShapeDtypeStruct((batch, n_pos, D_MODEL), jnp.float32),
    )


@jax.jit
def kernel(x, embed_table, pe):
    batch, seq = x.shape
    x_flat = x.reshape(batch * seq).astype(jnp.int32)
    pe2d = pe.reshape(pe.shape[1], pe.shape[2])[:seq]
    sc_out = _make_sc_kernel(batch, seq)(x_flat, embed_table, pe2d)
    sc_out = sc_out.reshape(batch, seq, D_MODEL)
    tc_part = _make_tc_kernel(batch, seq)(x_flat, embed_table, pe2d)
    return lax.dynamic_update_slice(sc_out, tc_part, (0, S_SC, 0))


# speedup vs baseline: 1.2318x; 1.2318x over previous
from math import sqrt

import jax
import jax.numpy as jnp
from jax import lax
from jax.experimental import pallas as pl
from jax.experimental.pallas import tpu as pltpu
from jax.experimental.pallas import tpu_sc as plsc

D_MODEL = 1024
SCALE = sqrt(D_MODEL)
NW = 32
LANES = 16
PCHUNK = 8
S_SC = 1536
K_TC = 128


def _make_sc_kernel(batch, seq):
    pos_per_w = S_SC // NW
    n_chunks = pos_per_w // PCHUNK
    mesh = plsc.VectorSubcoreMesh(core_axis_name="c", subcore_axis_name="s")

    @pl.kernel(
        out_type=jax.ShapeDtypeStruct((batch * seq, D_MODEL), jnp.float32),
        mesh=mesh,
        scratch_types=[
            pltpu.VMEM((batch * pos_per_w,), jnp.int32),
            pltpu.VMEM((batch, PCHUNK, D_MODEL), jnp.float32),
            pltpu.VMEM((batch, PCHUNK, D_MODEL), jnp.float32),
            pltpu.VMEM((batch, PCHUNK, D_MODEL), jnp.float32),
            pltpu.VMEM((PCHUNK, D_MODEL), jnp.float32),
            pltpu.VMEM((PCHUNK, D_MODEL), jnp.float32),
            pltpu.VMEM((PCHUNK, D_MODEL), jnp.float32),
            pltpu.SemaphoreType.DMA,
            pltpu.SemaphoreType.DMA,
            pltpu.SemaphoreType.DMA,
            pltpu.SemaphoreType.DMA,
        ],
    )
    def emb_kernel(
        x_hbm, table_hbm, pe_hbm, out_hbm,
        idx_v, rows0, rows1, rows2, pe0, pe1, pe2,
        sem0, sem1, sem2, sem_out,
    ):
        rows = (rows0, rows1, rows2)
        pes = (pe0, pe1, pe2)
        sems = (sem0, sem1, sem2)

        wid = lax.axis_index("s") * 2 + lax.axis_index("c")
        p0 = wid * pos_per_w

        idx_copies = [
            pltpu.async_copy(
                x_hbm.at[pl.ds(b * seq + p0, pos_per_w)],
                idx_v.at[pl.ds(b * pos_per_w, pos_per_w)],
                sem0,
            )
            for b in range(batch)
        ]
        for cp in idx_copies:
            cp.wait()

        def fire_in(c, k):
            off = c * PCHUNK
            cps = [
                pltpu.async_copy(
                    table_hbm.at[idx_v.at[pl.ds(b * pos_per_w + off, PCHUNK)]],
                    rows[k].at[b],
                    sems[k],
                )
                for b in range(batch)
            ]
            cps.append(
                pltpu.async_copy(pe_hbm.at[pl.ds(p0 + off, PCHUNK)], pes[k], sems[k])
            )
            return cps

        def fire_stores(c, k):
            off = c * PCHUNK
            return [
                pltpu.async_copy(
                    rows[k].at[b],
                    out_hbm.at[pl.ds(b * seq + p0 + off, PCHUNK)],
                    sem_out,
                )
                for b in range(batch)
            ]

        def compute(k):
            def row_body(r, carry):
                def col_body(j, carry2):
                    sl = pl.ds(j * LANES, LANES)
                    pe_reg = pes[k][r, sl]
                    for b in range(batch):
                        rows[k][b, r, sl] = rows[k][b, r, sl] * SCALE + pe_reg
                    return carry2

                return lax.fori_loop(
                    0, D_MODEL // LANES, col_body, carry, unroll=8
                )

            lax.fori_loop(0, PCHUNK, row_body, 0)

        nbuf = 3
        in_flight = {0: fire_in(0, 0), 1: fire_in(1, 1)}
        store_flight = {}
        for c in range(n_chunks):
            k = c % nbuf
            if c + 2 < n_chunks:
                for cp in store_flight.pop(c - 1, ()):
                    cp.wait()
                in_flight[c + 2] = fire_in(c + 2, (c + 2) % nbuf)
            for cp in in_flight.pop(c):
                cp.wait()
            compute(k)
            store_flight[c] = fire_stores(c, k)
        for cps in store_flight.values():
            for cp in cps:
                cp.wait()

    return emb_kernel


def _make_tc_kernel(batch, seq):
    n_pos = seq - S_SC
    steps = n_pos // K_TC
    n_steps = batch * steps

    def body(idx_sm, table_hbm, pe_vm, out_vm, buf, sem0, sem1):
        b = pl.program_id(0)
        i = pl.program_id(1)
        t = b * steps + i
        sems = (sem0, sem1)

        def fire(t2, slot):
            b2 = t2 // steps
            i2 = t2 % steps
            base = b2 * seq + S_SC + i2 * K_TC
            for j in range(K_TC):
                pltpu.make_async_copy(
                    table_hbm.at[pl.ds(idx_sm[base + j], 1)],
                    buf.at[pl.ds(slot * K_TC + j, 1)],
                    sems[slot],
                ).start()

        def drain(slot):
            for j in range(K_TC):
                pltpu.make_async_copy(
                    table_hbm.at[pl.ds(0, 1)],
                    buf.at[pl.ds(slot * K_TC + j, 1)],
                    sems[slot],
                ).wait()

        parity = t % 2

        @pl.when(t == 0)
        def _prime():
            fire(t, 0)

        @pl.when((t + 1 < n_steps) & (parity == 0))
        def _prefetch_odd():
            fire(t + 1, 1)

        @pl.when((t + 1 < n_steps) & (parity == 1))
        def _prefetch_even():
            fire(t + 1, 0)

        @pl.when(parity == 0)
        def _drain0():
            drain(0)

        @pl.when(parity == 1)
        def _drain1():
            drain(1)

        out_vm[0] = buf[pl.ds(parity * K_TC, K_TC)] * SCALE + pe_vm[...]

    grid_spec = pltpu.PrefetchScalarGridSpec(
        num_scalar_prefetch=1,
        grid=(batch, steps),
        in_specs=[
            pl.BlockSpec(memory_space=pltpu.MemorySpace.HBM),
            pl.BlockSpec((K_TC, D_MODEL), lambda b, i, idx: (S_SC // K_TC + i, 0)),
        ],
        out_specs=pl.BlockSpec((1, K_TC, D_MODEL), lambda b, i, idx: (b, i, 0)),
        scratch_shapes=[
            pltpu.VMEM((2 * K_TC, D_MODEL), jnp.float32),
            pltpu.SemaphoreType.DMA,
            pltpu.SemaphoreType.DMA,
        ],
    )
    return pl.pallas_call(
        body,
        grid_spec=grid_spec,
        out_shape=jax.ShapeDtypeStruct((batch, n_pos, D_MODEL), jnp.float32),
    )


@jax.jit
def kernel(x, embed_table, pe):
    batch, seq = x.shape
    x_flat = x.reshape(batch * seq).astype(jnp.int32)
    pe2d = pe.reshape(pe.shape[1], pe.shape[2])[:seq]
    sc_out = _make_sc_kernel(batch, seq)(x_flat, embed_table, pe2d)
    sc_out = sc_out.reshape(batch, seq, D_MODEL)
    tc_part = _make_tc_kernel(batch, seq)(x_flat, embed_table, pe2d)
    return lax.dynamic_update_slice(sc_out, tc_part, (0, S_SC, 0))


# speedup vs baseline: 1.3890x; 1.1276x over previous
from math import sqrt

import jax
import jax.numpy as jnp
from jax import lax
from jax.experimental import pallas as pl
from jax.experimental.pallas import tpu as pltpu
from jax.experimental.pallas import tpu_sc as plsc

D_MODEL = 1024
SCALE = sqrt(D_MODEL)
NW = 32
LANES = 16
PCHUNK = 8


def _make_sc_kernel(batch, seq):
    pos_per_w = seq // NW
    n_chunks = pos_per_w // PCHUNK
    mesh = plsc.VectorSubcoreMesh(core_axis_name="c", subcore_axis_name="s")

    @pl.kernel(
        out_type=jax.ShapeDtypeStruct((batch * seq, D_MODEL), jnp.float32),
        mesh=mesh,
        scratch_types=[
            pltpu.VMEM((batch * pos_per_w,), jnp.int32),
            pltpu.VMEM((batch, PCHUNK, D_MODEL), jnp.float32),
            pltpu.VMEM((batch, PCHUNK, D_MODEL), jnp.float32),
            pltpu.VMEM((batch, PCHUNK, D_MODEL), jnp.float32),
            pltpu.VMEM((PCHUNK, D_MODEL), jnp.float32),
            pltpu.VMEM((PCHUNK, D_MODEL), jnp.float32),
            pltpu.VMEM((PCHUNK, D_MODEL), jnp.float32),
            pltpu.SemaphoreType.DMA,
            pltpu.SemaphoreType.DMA,
            pltpu.SemaphoreType.DMA,
            pltpu.SemaphoreType.DMA,
        ],
    )
    def emb_kernel(
        x_hbm, table_hbm, pe_hbm, out_hbm,
        idx_v, rows0, rows1, rows2, pe0, pe1, pe2,
        sem0, sem1, sem2, sem_out,
    ):
        rows = (rows0, rows1, rows2)
        pes = (pe0, pe1, pe2)
        sems = (sem0, sem1, sem2)

        wid = lax.axis_index("s") * 2 + lax.axis_index("c")
        p0 = wid * pos_per_w

        def fire_pe(c, k):
            off = c * PCHUNK
            return pltpu.async_copy(
                pe_hbm.at[pl.ds(p0 + off, PCHUNK)], pes[k], sems[k]
            )

        def fire_gathers(c, k):
            off = c * PCHUNK
            return [
                pltpu.async_copy(
                    table_hbm.at[idx_v.at[pl.ds(b * pos_per_w + off, PCHUNK)]],
                    rows[k].at[b],
                    sems[k],
                )
                for b in range(batch)
            ]

        def fire_in(c, k):
            return fire_gathers(c, k) + [fire_pe(c, k)]

        pe_head = [fire_pe(0, 0), fire_pe(1, 1)]
        idx_copies = [
            pltpu.async_copy(
                x_hbm.at[pl.ds(b * seq + p0, pos_per_w)],
                idx_v.at[pl.ds(b * pos_per_w, pos_per_w)],
                sem_out,
            )
            for b in range(batch)
        ]
        for cp in idx_copies:
            cp.wait()

        def fire_stores(c, k):
            off = c * PCHUNK
            return [
                pltpu.async_copy(
                    rows[k].at[b],
                    out_hbm.at[pl.ds(b * seq + p0 + off, PCHUNK)],
                    sem_out,
                )
                for b in range(batch)
            ]

        def compute(k):
            def row_body(r, carry):
                def col_body(j, carry2):
                    sl = pl.ds(j * LANES, LANES)
                    pe_reg = pes[k][r, sl]
                    for b in range(batch):
                        rows[k][b, r, sl] = rows[k][b, r, sl] * SCALE + pe_reg
                    return carry2

                return lax.fori_loop(
                    0, D_MODEL // LANES, col_body, carry, unroll=8
                )

            lax.fori_loop(0, PCHUNK, row_body, 0)

        nbuf = 3
        in_flight = {
            0: fire_gathers(0, 0) + [pe_head[0]],
            1: fire_gathers(1, 1) + [pe_head[1]],
        }
        store_flight = {}
        for c in range(n_chunks):
            k = c % nbuf
            if c + 2 < n_chunks:
                for cp in store_flight.pop(c - 1, ()):
                    cp.wait()
                in_flight[c + 2] = fire_in(c + 2, (c + 2) % nbuf)
            for cp in in_flight.pop(c):
                cp.wait()
            compute(k)
            store_flight[c] = fire_stores(c, k)
        for cps in store_flight.values():
            for cp in cps:
                cp.wait()

    return emb_kernel


@jax.jit
def kernel(x, embed_table, pe):
    batch, seq = x.shape
    x_flat = x.reshape(batch * seq).astype(jnp.int32)
    pe2d = pe.reshape(pe.shape[1], pe.shape[2])[:seq]
    out = _make_sc_kernel(batch, seq)(x_flat, embed_table, pe2d)
    return out.reshape(batch, seq, D_MODEL)


# speedup vs baseline: 1.3933x; 1.0031x over previous
from math import sqrt

import jax
import jax.numpy as jnp
from jax import lax
from jax.experimental import pallas as pl
from jax.experimental.pallas import tpu as pltpu
from jax.experimental.pallas import tpu_sc as plsc

D_MODEL = 1024
SCALE = sqrt(D_MODEL)
NW = 32
LANES = 16
PCHUNK = 8


def _make_sc_kernel(batch, seq):
    pos_per_w = seq // NW
    n_chunks = pos_per_w // PCHUNK
    mesh = plsc.VectorSubcoreMesh(core_axis_name="c", subcore_axis_name="s")

    @pl.kernel(
        out_type=jax.ShapeDtypeStruct((batch * seq, D_MODEL), jnp.float32),
        mesh=mesh,
        scratch_types=[
            pltpu.VMEM((batch * pos_per_w,), jnp.int32),
            pltpu.VMEM((batch, PCHUNK, D_MODEL), jnp.float32),
            pltpu.VMEM((batch, PCHUNK, D_MODEL), jnp.float32),
            pltpu.VMEM((batch, PCHUNK, D_MODEL), jnp.float32),
            pltpu.VMEM((PCHUNK, D_MODEL), jnp.float32),
            pltpu.VMEM((PCHUNK, D_MODEL), jnp.float32),
            pltpu.VMEM((PCHUNK, D_MODEL), jnp.float32),
            pltpu.SemaphoreType.DMA,
            pltpu.SemaphoreType.DMA,
            pltpu.SemaphoreType.DMA,
            pltpu.SemaphoreType.DMA,
        ],
    )
    def emb_kernel(
        x_hbm, table_hbm, pe_hbm, out_hbm,
        idx_v, rows0, rows1, rows2, pe0, pe1, pe2,
        sem0, sem1, sem2, sem_out,
    ):
        rows = (rows0, rows1, rows2)
        pes = (pe0, pe1, pe2)
        sems = (sem0, sem1, sem2)

        wid = lax.axis_index("s") * 2 + lax.axis_index("c")
        p0 = wid * pos_per_w

        def fire_pe(c, k):
            off = c * PCHUNK
            return pltpu.async_copy(
                pe_hbm.at[pl.ds(p0 + off, PCHUNK)], pes[k], sems[k]
            )

        def fire_gathers(c, k):
            off = c * PCHUNK
            return [
                pltpu.async_copy(
                    table_hbm.at[idx_v.at[pl.ds(b * pos_per_w + off, PCHUNK)]],
                    rows[k].at[b],
                    sems[k],
                )
                for b in range(batch)
            ]

        def fire_in(c, k):
            return fire_gathers(c, k) + [fire_pe(c, k)]

        pe_head = [fire_pe(0, 0), fire_pe(1, 1)]
        idx_copies = [
            pltpu.async_copy(
                x_hbm.at[b, pl.ds(p0, pos_per_w)],
                idx_v.at[pl.ds(b * pos_per_w, pos_per_w)],
                sem_out,
            )
            for b in range(batch)
        ]
        for cp in idx_copies:
            cp.wait()

        def fire_stores(c, k):
            off = c * PCHUNK
            return [
                pltpu.async_copy(
                    rows[k].at[b],
                    out_hbm.at[pl.ds(b * seq + p0 + off, PCHUNK)],
                    sem_out,
                )
                for b in range(batch)
            ]

        def compute(k):
            def row_body(r, carry):
                def col_body(j, carry2):
                    sl = pl.ds(j * LANES, LANES)
                    pe_reg = pes[k][r, sl]
                    for b in range(batch):
                        rows[k][b, r, sl] = rows[k][b, r, sl] * SCALE + pe_reg
                    return carry2

                return lax.fori_loop(
                    0, D_MODEL // LANES, col_body, carry, unroll=8
                )

            lax.fori_loop(0, PCHUNK, row_body, 0)

        nbuf = 3
        in_flight = {
            0: fire_gathers(0, 0) + [pe_head[0]],
            1: fire_gathers(1, 1) + [pe_head[1]],
        }
        store_flight = {}
        for c in range(n_chunks):
            k = c % nbuf
            if c + 2 < n_chunks:
                for cp in store_flight.pop(c - 1, ()):
                    cp.wait()
                in_flight[c + 2] = fire_in(c + 2, (c + 2) % nbuf)
            for cp in in_flight.pop(c):
                cp.wait()
            compute(k)
            store_flight[c] = fire_stores(c, k)
        for cps in store_flight.values():
            for cp in cps:
                cp.wait()

    return emb_kernel


@jax.jit
def kernel(x, embed_table, pe):
    batch, seq = x.shape
    x2d = x.astype(jnp.int32)
    pe2d = pe[0, :seq]
    out = _make_sc_kernel(batch, seq)(x2d, embed_table, pe2d)
    return out.reshape(batch, seq, D_MODEL)


# speedup vs baseline: 1.4041x; 1.0077x over previous
from math import sqrt

import jax
import jax.numpy as jnp
from jax import lax
from jax.experimental import pallas as pl
from jax.experimental.pallas import tpu as pltpu
from jax.experimental.pallas import tpu_sc as plsc

D_MODEL = 1024
SCALE = sqrt(D_MODEL)
NW = 32
LANES = 16
PCHUNK = 8


def _make_sc_kernel(batch, seq):
    pos_per_w = seq // NW
    n_chunks = pos_per_w // PCHUNK
    mesh = plsc.VectorSubcoreMesh(core_axis_name="c", subcore_axis_name="s")

    @pl.kernel(
        out_type=jax.ShapeDtypeStruct((batch * seq, D_MODEL), jnp.float32),
        mesh=mesh,
        scratch_types=[
            pltpu.VMEM((batch * pos_per_w,), jnp.int32),
            pltpu.VMEM((batch, PCHUNK, D_MODEL), jnp.float32),
            pltpu.VMEM((batch, PCHUNK, D_MODEL), jnp.float32),
            pltpu.VMEM((batch, PCHUNK, D_MODEL), jnp.float32),
            pltpu.VMEM((PCHUNK, D_MODEL), jnp.float32),
            pltpu.VMEM((PCHUNK, D_MODEL), jnp.float32),
            pltpu.VMEM((PCHUNK, D_MODEL), jnp.float32),
            pltpu.SemaphoreType.DMA,
            pltpu.SemaphoreType.DMA,
            pltpu.SemaphoreType.DMA,
            pltpu.SemaphoreType.DMA,
        ],
    )
    def emb_kernel(
        x_hbm, table_hbm, pe_hbm, out_hbm,
        idx_v, rows0, rows1, rows2, pe0, pe1, pe2,
        sem0, sem1, sem2, sem_out,
    ):
        rows = (rows0, rows1, rows2)
        pes = (pe0, pe1, pe2)
        sems = (sem0, sem1, sem2)

        wid = lax.axis_index("s") * 2 + lax.axis_index("c")
        p0 = wid * pos_per_w

        def fire_pe(c, k):
            off = c * PCHUNK
            return pltpu.async_copy(
                pe_hbm.at[pl.ds(p0 + off, PCHUNK)], pes[k], sems[k]
            )

        def fire_gathers(c, k):
            off = c * PCHUNK
            return [
                pltpu.async_copy(
                    table_hbm.at[idx_v.at[pl.ds(b * pos_per_w + off, PCHUNK)]],
                    rows[k].at[b],
                    sems[k],
                )
                for b in range(batch)
            ]

        def fire_in(c, k):
            return fire_gathers(c, k) + [fire_pe(c, k)]

        pe_head = [fire_pe(0, 0), fire_pe(1, 1)]
        idx_copies = [
            pltpu.async_copy(
                x_hbm.at[b, pl.ds(p0, pos_per_w)],
                idx_v.at[pl.ds(b * pos_per_w, pos_per_w)],
                sem_out,
            )
            for b in range(batch)
        ]
        for cp in idx_copies:
            cp.wait()

        def fire_stores(c, k):
            off = c * PCHUNK
            return [
                pltpu.async_copy(
                    rows[k].at[b],
                    out_hbm.at[pl.ds(b * seq + p0 + off, PCHUNK)],
                    sem_out,
                )
                for b in range(batch)
            ]

        def compute(k):
            def row_body(r, carry):
                def col_body(j, carry2):
                    sl = pl.ds(j * LANES, LANES)
                    pe_reg = pes[k][r, sl]
                    for b in range(batch):
                        rows[k][b, r, sl] = rows[k][b, r, sl] * SCALE + pe_reg
                    return carry2

                return lax.fori_loop(
                    0, D_MODEL // LANES, col_body, carry, unroll=8
                )

            lax.fori_loop(0, PCHUNK, row_body, 0)

        nbuf = 3
        in_flight = {
            0: fire_gathers(0, 0) + [pe_head[0]],
            1: fire_gathers(1, 1) + [pe_head[1]],
        }
        store_flight = {}
        for c in range(n_chunks):
            k = c % nbuf
            if c + 2 < n_chunks:
                k2 = (c + 2) % nbuf
                pe_cp = fire_pe(c + 2, k2)
                for cp in store_flight.pop(c - 1, ()):
                    cp.wait()
                in_flight[c + 2] = fire_gathers(c + 2, k2) + [pe_cp]
            for cp in in_flight.pop(c):
                cp.wait()
            compute(k)
            store_flight[c] = fire_stores(c, k)
        for cps in store_flight.values():
            for cp in cps:
                cp.wait()

    return emb_kernel


@jax.jit
def kernel(x, embed_table, pe):
    batch, seq = x.shape
    x2d = x.astype(jnp.int32)
    pe2d = pe[0, :seq]
    out = _make_sc_kernel(batch, seq)(x2d, embed_table, pe2d)
    return out.reshape(batch, seq, D_MODEL)
